# baseline (speedup 1.0000x reference)
import functools

import jax
import jax.numpy as jnp
from jax import lax
from jax.experimental import pallas as pl
from jax.experimental.pallas import tpu as pltpu
from jax.experimental.pallas import tpu_sc as plsc

_MAJORITY_CLASS = 0
_NUM_CLASSES = 2


@functools.lru_cache(maxsize=None)
def _build_fill(batch: int):
    info = plsc.get_sparse_core_info()
    ns, lanes = info.num_subcores, info.num_lanes
    total = batch * _NUM_CLASSES
    assert total % ns == 0
    chunk = total // ns
    assert chunk % lanes == 0 and chunk % 8 == 0

    mesh = plsc.VectorSubcoreMesh(
        core_axis_name="c", subcore_axis_name="s", num_cores=1
    )

    @functools.partial(
        pl.kernel,
        mesh=mesh,
        out_type=jax.ShapeDtypeStruct((total,), jnp.float32),
        scratch_types=[pltpu.VMEM((chunk,), jnp.float32)],
    )
    def fill(out_hbm, buf):
        base = lax.axis_index("s") * chunk
        lane = lax.iota(jnp.int32, lanes)
        pattern = jnp.where(
            lax.rem(lane, _NUM_CLASSES) == _MAJORITY_CLASS,
            jnp.float32(0.0),
            jnp.float32(-jnp.inf),
        )

        def body(i, _):
            buf[pl.ds(i * lanes, lanes)] = pattern
            return 0

        lax.fori_loop(0, chunk // lanes, body, 0)
        pltpu.sync_copy(buf, out_hbm.at[pl.ds(base, chunk)])

    @jax.jit
    def run():
        return fill().reshape(batch, _NUM_CLASSES)

    return run


def kernel(x):
    return _build_fill(x.shape[0])()
